# Initial kernel scaffold; baseline (speedup 1.0000x reference)
import jax
import jax.numpy as jnp
from jax.experimental import pallas as pl


def kernel(x, edge_index, W_gcn, b_gcn, W_out, b_out):
    raise NotImplementedError("write your pallas kernel here")


# speedup vs baseline: 103.6849x; 103.6849x over previous
import functools

import jax
import jax.numpy as jnp
from jax import lax
from jax.experimental import pallas as pl
from jax.experimental.pallas import tpu as pltpu
from jax.experimental.pallas import tpu_sc as plsc

N = 10000
E = 320000
F = 128
NC = 2
NS = 16
NW = NC * NS
EPW = E // NW
L = 16

@functools.cache
def _sc_mesh():
    return plsc.VectorSubcoreMesh(core_axis_name="c", subcore_axis_name="s",
                                  num_cores=NC, num_subcores=NS)


@functools.cache
def _hist_sc_kernel():
    return pl.kernel(
        _hist_sc_body,
        out_type=jax.ShapeDtypeStruct((NW, N), jnp.float32),
        mesh=_sc_mesh(),
        scratch_types=[
            pltpu.VMEM((EPW,), jnp.int32),
            pltpu.VMEM((N,), jnp.float32),
        ],
        compiler_params=pltpu.CompilerParams(needs_layout_passes=False),
    )


def _hist_sc_body(dst_hbm, out_hbm, dst_v, hist_v):
    wid = lax.axis_index("s") * NC + lax.axis_index("c")
    pltpu.sync_copy(dst_hbm.at[pl.ds(wid * EPW, EPW)], dst_v)

    z16 = jnp.zeros((L,), jnp.float32)

    def zero_body(i, c):
        hist_v[pl.ds(i * L, L)] = z16
        return c

    lax.fori_loop(0, N // L, zero_body, 0, unroll=4)

    ones = jnp.ones((L,), jnp.float32)

    def body(i, c):
        t = dst_v[pl.ds(i * L, L)]
        plsc.addupdate_scatter(hist_v, [t], ones)
        return c

    lax.fori_loop(0, EPW // L, body, 0, unroll=4)

    pltpu.sync_copy(hist_v, out_hbm.at[wid])


@functools.cache
def _msg_sc_kernel():
    return pl.kernel(
        _msg_sc_body,
        out_type=jax.ShapeDtypeStruct((NW * 3, N), jnp.float32),
        mesh=_sc_mesh(),
        scratch_types=[
            pltpu.VMEM((EPW,), jnp.int32),
            pltpu.VMEM((EPW,), jnp.int32),
            pltpu.VMEM((N,), jnp.float32),
            pltpu.VMEM((N,), jnp.float32),
            pltpu.VMEM((N,), jnp.float32),
            pltpu.VMEM((N,), jnp.float32),
            pltpu.VMEM((N,), jnp.float32),
            pltpu.VMEM((N,), jnp.float32),
        ],
        compiler_params=pltpu.CompilerParams(needs_layout_passes=False),
    )


def _msg_sc_body(src_hbm, dst_hbm, y0_hbm, y1_hbm, y2_hbm, out_hbm,
            src_v, dst_v, y0, y1, y2, a0, a1, a2):
    wid = lax.axis_index("s") * NC + lax.axis_index("c")
    pltpu.sync_copy(src_hbm.at[pl.ds(wid * EPW, EPW)], src_v)
    pltpu.sync_copy(dst_hbm.at[pl.ds(wid * EPW, EPW)], dst_v)
    pltpu.sync_copy(y0_hbm, y0)
    pltpu.sync_copy(y1_hbm, y1)
    pltpu.sync_copy(y2_hbm, y2)

    z16 = jnp.zeros((L,), jnp.float32)

    def zero_body(i, c):
        a0[pl.ds(i * L, L)] = z16
        a1[pl.ds(i * L, L)] = z16
        a2[pl.ds(i * L, L)] = z16
        return c

    lax.fori_loop(0, N // L, zero_body, 0, unroll=4)

    def body(i, c):
        s = src_v[pl.ds(i * L, L)]
        t = dst_v[pl.ds(i * L, L)]
        plsc.addupdate_scatter(a0, [t], plsc.load_gather(y0, [s]))
        plsc.addupdate_scatter(a1, [t], plsc.load_gather(y1, [s]))
        plsc.addupdate_scatter(a2, [t], plsc.load_gather(y2, [s]))
        return c

    lax.fori_loop(0, EPW // L, body, 0, unroll=4)

    pltpu.sync_copy(a0, out_hbm.at[wid * 3 + 0])
    pltpu.sync_copy(a1, out_hbm.at[wid * 3 + 1])
    pltpu.sync_copy(a2, out_hbm.at[wid * 3 + 2])


def _mid_tc_body(hp_ref, xT_ref, w8t_ref, y_ref, d_ref):
    deg = jnp.sum(hp_ref[...], axis=0) + 1.0
    dv = lax.rsqrt(deg)
    xw = jnp.dot(w8t_ref[...], xT_ref[...],
                 preferred_element_type=jnp.float32)
    y_ref[...] = xw * dv[None, :]
    d_ref[...] = dv[None, :]


_mid_tc = pl.pallas_call(
    _mid_tc_body,
    out_shape=(jax.ShapeDtypeStruct((8, N), jnp.float32),
               jax.ShapeDtypeStruct((1, N), jnp.float32)),
)


def _fin_tc_body(ap_ref, sel_ref, y_ref, d_ref, b3_ref, wo_ref, bo_ref,
                 h_ref, z_ref):
    acc = jnp.dot(sel_ref[...], ap_ref[...],
                  preferred_element_type=jnp.float32)
    tot = acc + y_ref[0:3, :]
    hT = jnp.maximum(tot * d_ref[...] + b3_ref[...], 0.0)
    h_ref[...] = hT
    z_ref[...] = jnp.dot(wo_ref[...], hT,
                         preferred_element_type=jnp.float32) + bo_ref[...]


_fin_tc = pl.pallas_call(
    _fin_tc_body,
    out_shape=(jax.ShapeDtypeStruct((3, N), jnp.float32),
               jax.ShapeDtypeStruct((7, N), jnp.float32)),
)


def kernel(x, edge_index, W_gcn, b_gcn, W_out, b_out):
    src = edge_index[0]
    dst = edge_index[1]
    xT = x.T
    w8t = jnp.zeros((8, F), jnp.float32).at[:3, :].set(W_gcn.T)

    hp = _hist_sc_kernel()(dst)
    yT8, drow = _mid_tc(hp, xT, w8t)
    ap = _msg_sc_kernel()(src, dst, yT8[0], yT8[1], yT8[2])
    sel = jnp.tile(jnp.eye(3, dtype=jnp.float32), (1, NW))
    hT, zT = _fin_tc(ap, sel, yT8, drow,
                     b_gcn.reshape(3, 1), W_out.T, b_out.reshape(7, 1))
    return hT.T, zT.T


# speedup vs baseline: 105.3759x; 1.0163x over previous
import functools

import jax
import jax.numpy as jnp
from jax import lax
from jax.experimental import pallas as pl
from jax.experimental.pallas import tpu as pltpu
from jax.experimental.pallas import tpu_sc as plsc

N = 10000
E = 320000
F = 128
NC = 2
NS = 16
NW = NC * NS
EPW = E // NW
L = 16
UNROLL = 5


@functools.cache
def _sc_mesh():
    return plsc.VectorSubcoreMesh(core_axis_name="c", subcore_axis_name="s",
                                  num_cores=NC, num_subcores=NS)


@functools.cache
def _hist_sc_kernel():
    return pl.kernel(
        _hist_sc_body,
        out_type=jax.ShapeDtypeStruct((NW, N), jnp.float32),
        mesh=_sc_mesh(),
        scratch_types=[
            pltpu.VMEM((EPW,), jnp.int32),
            pltpu.VMEM((N,), jnp.float32),
        ],
        compiler_params=pltpu.CompilerParams(needs_layout_passes=False),
    )


def _hist_sc_body(dst_hbm, out_hbm, dst_v, hist_v):
    wid = lax.axis_index("s") * NC + lax.axis_index("c")
    pltpu.sync_copy(dst_hbm.at[pl.ds(wid * EPW, EPW)], dst_v)

    z16 = jnp.zeros((L,), jnp.float32)

    @plsc.parallel_loop(0, N // L, unroll=UNROLL)
    def _(i):
        hist_v[pl.ds(i * L, L)] = z16

    ones = jnp.ones((L,), jnp.float32)

    @plsc.parallel_loop(0, EPW // L, unroll=UNROLL)
    def _(i):
        t = dst_v[pl.ds(i * L, L)]
        plsc.addupdate_scatter(hist_v, [t], ones)

    pltpu.sync_copy(hist_v, out_hbm.at[wid])


@functools.cache
def _msg_sc_kernel():
    return pl.kernel(
        _msg_sc_body,
        out_type=jax.ShapeDtypeStruct((NW * 3, N), jnp.float32),
        mesh=_sc_mesh(),
        scratch_types=[
            pltpu.VMEM((EPW,), jnp.int32),
            pltpu.VMEM((EPW,), jnp.int32),
            pltpu.VMEM((N,), jnp.float32),
            pltpu.VMEM((N,), jnp.float32),
            pltpu.VMEM((N,), jnp.float32),
            pltpu.VMEM((N,), jnp.float32),
            pltpu.VMEM((N,), jnp.float32),
            pltpu.VMEM((N,), jnp.float32),
        ],
        compiler_params=pltpu.CompilerParams(needs_layout_passes=False),
    )


def _msg_sc_body(src_hbm, dst_hbm, y0_hbm, y1_hbm, y2_hbm, out_hbm,
                 src_v, dst_v, y0, y1, y2, a0, a1, a2):
    wid = lax.axis_index("s") * NC + lax.axis_index("c")
    pltpu.sync_copy(src_hbm.at[pl.ds(wid * EPW, EPW)], src_v)
    pltpu.sync_copy(dst_hbm.at[pl.ds(wid * EPW, EPW)], dst_v)
    pltpu.sync_copy(y0_hbm, y0)
    pltpu.sync_copy(y1_hbm, y1)
    pltpu.sync_copy(y2_hbm, y2)

    z16 = jnp.zeros((L,), jnp.float32)

    @plsc.parallel_loop(0, N // L, unroll=UNROLL)
    def _(i):
        a0[pl.ds(i * L, L)] = z16
        a1[pl.ds(i * L, L)] = z16
        a2[pl.ds(i * L, L)] = z16

    @plsc.parallel_loop(0, EPW // L, unroll=UNROLL)
    def _(i):
        s = src_v[pl.ds(i * L, L)]
        t = dst_v[pl.ds(i * L, L)]
        plsc.addupdate_scatter(a0, [t], plsc.load_gather(y0, [s]))
        plsc.addupdate_scatter(a1, [t], plsc.load_gather(y1, [s]))
        plsc.addupdate_scatter(a2, [t], plsc.load_gather(y2, [s]))

    pltpu.sync_copy(a0, out_hbm.at[wid * 3 + 0])
    pltpu.sync_copy(a1, out_hbm.at[wid * 3 + 1])
    pltpu.sync_copy(a2, out_hbm.at[wid * 3 + 2])


def _mid_tc_body(hp_ref, x_ref, w8_ref, y_ref, y0_ref, y1_ref, y2_ref, d_ref):
    deg = jnp.sum(hp_ref[...], axis=0) + 1.0
    dv = lax.rsqrt(deg)
    xw = jnp.dot(x_ref[...], w8_ref[...],
                 preferred_element_type=jnp.float32)
    yT = xw.T[0:3, :] * dv[None, :]
    y_ref[...] = yT
    y0_ref[...] = yT[0]
    y1_ref[...] = yT[1]
    y2_ref[...] = yT[2]
    d_ref[...] = dv[None, :]


_mid_tc = pl.pallas_call(
    _mid_tc_body,
    out_shape=(jax.ShapeDtypeStruct((3, N), jnp.float32),
               jax.ShapeDtypeStruct((N,), jnp.float32),
               jax.ShapeDtypeStruct((N,), jnp.float32),
               jax.ShapeDtypeStruct((N,), jnp.float32),
               jax.ShapeDtypeStruct((1, N), jnp.float32)),
)


def _fin_tc_body(ap_ref, sel_ref, y_ref, d_ref, b3_ref, wo_ref, bo_ref,
                 h_ref, z_ref):
    acc = jnp.dot(sel_ref[...], ap_ref[...],
                  preferred_element_type=jnp.float32)
    tot = acc + y_ref[...]
    hT = jnp.maximum(tot * d_ref[...] + b3_ref[...], 0.0)
    zT = jnp.dot(wo_ref[...], hT,
                 preferred_element_type=jnp.float32) + bo_ref[...]
    h_ref[...] = hT.T
    z_ref[...] = zT.T


_fin_tc = pl.pallas_call(
    _fin_tc_body,
    out_shape=(jax.ShapeDtypeStruct((N, 3), jnp.float32),
               jax.ShapeDtypeStruct((N, 7), jnp.float32)),
)


def kernel(x, edge_index, W_gcn, b_gcn, W_out, b_out):
    src = edge_index[0]
    dst = edge_index[1]
    w8 = jnp.zeros((F, 8), jnp.float32).at[:, :3].set(W_gcn)

    hp = _hist_sc_kernel()(dst)
    yT3, y0, y1, y2, drow = _mid_tc(hp, x, w8)
    ap = _msg_sc_kernel()(src, dst, y0, y1, y2)
    sel = jnp.tile(jnp.eye(3, dtype=jnp.float32), (1, NW))
    h, z = _fin_tc(ap, sel, yT3, drow,
                   b_gcn.reshape(3, 1), W_out.T, b_out.reshape(7, 1))
    return h, z


# speedup vs baseline: 123.7141x; 1.1740x over previous
import functools

import jax
import jax.numpy as jnp
from jax import lax
from jax.experimental import pallas as pl
from jax.experimental.pallas import tpu as pltpu
from jax.experimental.pallas import tpu_sc as plsc

N = 10000
E = 320000
F = 128
NC = 2
NS = 16
NW = NC * NS
EPW = E // NW
L = 16
UNROLL = 5
UNROLL_E = 25


@functools.cache
def _sc_mesh():
    return plsc.VectorSubcoreMesh(core_axis_name="c", subcore_axis_name="s",
                                  num_cores=NC, num_subcores=NS)


EW = EPW + 112


@functools.cache
def _hist_sc_kernel():
    return pl.kernel(
        _hist_sc_body,
        out_type=jax.ShapeDtypeStruct((NW, N), jnp.float32),
        mesh=_sc_mesh(),
        scratch_types=[
            pltpu.VMEM((2, EW), jnp.int32),
            pltpu.VMEM((N,), jnp.float32),
        ],
        compiler_params=pltpu.CompilerParams(needs_layout_passes=False),
    )


def _hist_sc_body(ei_hbm, out_hbm, ei_v, hist_v):
    wid = lax.axis_index("s") * NC + lax.axis_index("c")
    start = wid * EPW
    base = start // 128 * 128
    off = start - base
    pltpu.sync_copy(ei_hbm.at[:, pl.ds(base, EW)], ei_v)

    z16 = jnp.zeros((L,), jnp.float32)

    @plsc.parallel_loop(0, N // L, unroll=UNROLL)
    def _(i):
        hist_v[pl.ds(i * L, L)] = z16

    ones = jnp.ones((L,), jnp.float32)

    @plsc.parallel_loop(0, EPW // L, unroll=UNROLL_E)
    def _(i):
        t = ei_v[1, pl.ds(off + i * L, L)]
        plsc.addupdate_scatter(hist_v, [t], ones)

    pltpu.sync_copy(hist_v, out_hbm.at[wid])


@functools.cache
def _msg_sc_kernel():
    return pl.kernel(
        _msg_sc_body,
        out_type=jax.ShapeDtypeStruct((NW * 3, N), jnp.float32),
        mesh=_sc_mesh(),
        scratch_types=[
            pltpu.VMEM((2, EW), jnp.int32),
            pltpu.VMEM((N,), jnp.float32),
            pltpu.VMEM((N,), jnp.float32),
            pltpu.VMEM((N,), jnp.float32),
            pltpu.VMEM((N,), jnp.float32),
            pltpu.VMEM((N,), jnp.float32),
            pltpu.VMEM((N,), jnp.float32),
        ],
        compiler_params=pltpu.CompilerParams(needs_layout_passes=False),
    )


def _msg_sc_body(ei_hbm, y0_hbm, y1_hbm, y2_hbm, out_hbm,
                 ei_v, y0, y1, y2, a0, a1, a2):
    wid = lax.axis_index("s") * NC + lax.axis_index("c")
    start = wid * EPW
    base = start // 128 * 128
    off = start - base
    pltpu.sync_copy(ei_hbm.at[:, pl.ds(base, EW)], ei_v)
    pltpu.sync_copy(y0_hbm, y0)
    pltpu.sync_copy(y1_hbm, y1)
    pltpu.sync_copy(y2_hbm, y2)

    z16 = jnp.zeros((L,), jnp.float32)

    @plsc.parallel_loop(0, N // L, unroll=UNROLL)
    def _(i):
        a0[pl.ds(i * L, L)] = z16
        a1[pl.ds(i * L, L)] = z16
        a2[pl.ds(i * L, L)] = z16

    @plsc.parallel_loop(0, EPW // L, unroll=UNROLL_E)
    def _(i):
        s = ei_v[0, pl.ds(off + i * L, L)]
        t = ei_v[1, pl.ds(off + i * L, L)]
        plsc.addupdate_scatter(a0, [t], plsc.load_gather(y0, [s]))
        plsc.addupdate_scatter(a1, [t], plsc.load_gather(y1, [s]))
        plsc.addupdate_scatter(a2, [t], plsc.load_gather(y2, [s]))

    pltpu.sync_copy(a0, out_hbm.at[wid * 3 + 0])
    pltpu.sync_copy(a1, out_hbm.at[wid * 3 + 1])
    pltpu.sync_copy(a2, out_hbm.at[wid * 3 + 2])


def _mid_tc_body(hp_ref, x_ref, w8_ref, y_ref, y0_ref, y1_ref, y2_ref, d_ref):
    deg = jnp.sum(hp_ref[...], axis=0) + 1.0
    dv = lax.rsqrt(deg)
    xw = jnp.dot(x_ref[...], w8_ref[...],
                 preferred_element_type=jnp.float32)
    yT = xw.T[0:3, :] * dv[None, :]
    y_ref[...] = yT
    y0_ref[...] = yT[0]
    y1_ref[...] = yT[1]
    y2_ref[...] = yT[2]
    d_ref[...] = dv[None, :]


_mid_tc = pl.pallas_call(
    _mid_tc_body,
    out_shape=(jax.ShapeDtypeStruct((3, N), jnp.float32),
               jax.ShapeDtypeStruct((N,), jnp.float32),
               jax.ShapeDtypeStruct((N,), jnp.float32),
               jax.ShapeDtypeStruct((N,), jnp.float32),
               jax.ShapeDtypeStruct((1, N), jnp.float32)),
)


def _fin_tc_body(ap_ref, sel_ref, y_ref, d_ref, b3_ref, wo_ref, bo_ref,
                 h_ref, z_ref):
    acc = jnp.dot(sel_ref[...], ap_ref[...],
                  preferred_element_type=jnp.float32)
    tot = (acc + y_ref[...]) * d_ref[...]
    h = jnp.maximum(tot.T + b3_ref[...], 0.0)
    h_ref[...] = h
    z_ref[...] = jnp.dot(h, wo_ref[...],
                         preferred_element_type=jnp.float32) + bo_ref[...]


_fin_tc = pl.pallas_call(
    _fin_tc_body,
    out_shape=(jax.ShapeDtypeStruct((N, 3), jnp.float32),
               jax.ShapeDtypeStruct((N, 7), jnp.float32)),
)


def kernel(x, edge_index, W_gcn, b_gcn, W_out, b_out):
    w8 = jnp.zeros((F, 8), jnp.float32).at[:, :3].set(W_gcn)

    hp = _hist_sc_kernel()(edge_index)
    yT3, y0, y1, y2, drow = _mid_tc(hp, x, w8)
    ap = _msg_sc_kernel()(edge_index, y0, y1, y2)
    sel = jnp.tile(jnp.eye(3, dtype=jnp.float32), (1, NW))
    h, z = _fin_tc(ap, sel, yT3, drow,
                   b_gcn.reshape(1, 3), W_out, b_out.reshape(1, 7))
    return h, z


# speedup vs baseline: 127.8613x; 1.0335x over previous
import functools

import jax
import jax.numpy as jnp
from jax import lax
from jax.experimental import pallas as pl
from jax.experimental.pallas import tpu as pltpu
from jax.experimental.pallas import tpu_sc as plsc

N = 10000
E = 320000
F = 128
NC = 2
NS = 16
NW = NC * NS
EPW = E // NW
L = 16
UNROLL = 5
UNROLL_E = 25


@functools.cache
def _sc_mesh():
    return plsc.VectorSubcoreMesh(core_axis_name="c", subcore_axis_name="s",
                                  num_cores=NC, num_subcores=NS)


EW = EPW + 112


@functools.cache
def _hist_sc_kernel():
    return pl.kernel(
        _hist_sc_body,
        out_type=jax.ShapeDtypeStruct((NW, N), jnp.float32),
        mesh=_sc_mesh(),
        scratch_types=[
            pltpu.VMEM((2, EW), jnp.int32),
            pltpu.VMEM((N,), jnp.float32),
        ],
        compiler_params=pltpu.CompilerParams(needs_layout_passes=False),
    )


def _hist_sc_body(ei_hbm, out_hbm, ei_v, hist_v):
    wid = lax.axis_index("s") * NC + lax.axis_index("c")
    start = wid * EPW
    base = start // 128 * 128
    off = start - base
    pltpu.sync_copy(ei_hbm.at[:, pl.ds(base, EW)], ei_v)

    z16 = jnp.zeros((L,), jnp.float32)

    @plsc.parallel_loop(0, N // L, unroll=UNROLL)
    def _(i):
        hist_v[pl.ds(i * L, L)] = z16

    ones = jnp.ones((L,), jnp.float32)

    @plsc.parallel_loop(0, EPW // L, unroll=UNROLL_E)
    def _(i):
        t = ei_v[1, pl.ds(off + i * L, L)]
        plsc.addupdate_scatter(hist_v, [t], ones)

    pltpu.sync_copy(hist_v, out_hbm.at[wid])


@functools.cache
def _msg_sc_kernel():
    return pl.kernel(
        _msg_sc_body,
        out_type=jax.ShapeDtypeStruct((NW * 3, N), jnp.float32),
        mesh=_sc_mesh(),
        scratch_types=[
            pltpu.VMEM((2, EW), jnp.int32),
            pltpu.VMEM((N,), jnp.float32),
            pltpu.VMEM((N,), jnp.float32),
            pltpu.VMEM((N,), jnp.float32),
            pltpu.VMEM((N,), jnp.float32),
            pltpu.VMEM((N,), jnp.float32),
            pltpu.VMEM((N,), jnp.float32),
        ],
        compiler_params=pltpu.CompilerParams(needs_layout_passes=False),
    )


def _msg_sc_body(ei_hbm, y0_hbm, y1_hbm, y2_hbm, out_hbm,
                 ei_v, y0, y1, y2, a0, a1, a2):
    wid = lax.axis_index("s") * NC + lax.axis_index("c")
    start = wid * EPW
    base = start // 128 * 128
    off = start - base
    pltpu.sync_copy(ei_hbm.at[:, pl.ds(base, EW)], ei_v)
    pltpu.sync_copy(y0_hbm, y0)
    pltpu.sync_copy(y1_hbm, y1)
    pltpu.sync_copy(y2_hbm, y2)

    z16 = jnp.zeros((L,), jnp.float32)

    @plsc.parallel_loop(0, N // L, unroll=UNROLL)
    def _(i):
        a0[pl.ds(i * L, L)] = z16
        a1[pl.ds(i * L, L)] = z16
        a2[pl.ds(i * L, L)] = z16

    @plsc.parallel_loop(0, EPW // L, unroll=UNROLL_E)
    def _(i):
        s = ei_v[0, pl.ds(off + i * L, L)]
        t = ei_v[1, pl.ds(off + i * L, L)]
        plsc.addupdate_scatter(a0, [t], plsc.load_gather(y0, [s]))
        plsc.addupdate_scatter(a1, [t], plsc.load_gather(y1, [s]))
        plsc.addupdate_scatter(a2, [t], plsc.load_gather(y2, [s]))

    pltpu.sync_copy(a0, out_hbm.at[wid * 3 + 0])
    pltpu.sync_copy(a1, out_hbm.at[wid * 3 + 1])
    pltpu.sync_copy(a2, out_hbm.at[wid * 3 + 2])


def _xw_tc_body(x_ref, w8_ref, xwT_ref):
    xw = jnp.dot(x_ref[...], w8_ref[...],
                 preferred_element_type=jnp.float32)
    xwT_ref[...] = xw.T[0:3, :]


_xw_tc = pl.pallas_call(
    _xw_tc_body,
    out_shape=jax.ShapeDtypeStruct((3, N), jnp.float32),
)


def _mid_tc_body(hp_ref, xwT_ref, y_ref, y0_ref, y1_ref, y2_ref, d_ref):
    deg = jnp.sum(hp_ref[...], axis=0) + 1.0
    dv = lax.rsqrt(deg)
    yT = xwT_ref[...] * dv[None, :]
    y_ref[...] = yT
    y0_ref[...] = yT[0]
    y1_ref[...] = yT[1]
    y2_ref[...] = yT[2]
    d_ref[...] = dv[None, :]


_mid_tc = pl.pallas_call(
    _mid_tc_body,
    out_shape=(jax.ShapeDtypeStruct((3, N), jnp.float32),
               jax.ShapeDtypeStruct((N,), jnp.float32),
               jax.ShapeDtypeStruct((N,), jnp.float32),
               jax.ShapeDtypeStruct((N,), jnp.float32),
               jax.ShapeDtypeStruct((1, N), jnp.float32)),
)


def _fin_tc_body(ap_ref, sel_ref, y_ref, d_ref, b3_ref, wo_ref, bo_ref,
                 h_ref, z_ref):
    acc = jnp.dot(sel_ref[...], ap_ref[...],
                  preferred_element_type=jnp.float32)
    tot = (acc + y_ref[...]) * d_ref[...]
    h = jnp.maximum(tot.T + b3_ref[...], 0.0)
    h_ref[...] = h
    z_ref[...] = jnp.dot(h, wo_ref[...],
                         preferred_element_type=jnp.float32) + bo_ref[...]


_fin_tc = pl.pallas_call(
    _fin_tc_body,
    out_shape=(jax.ShapeDtypeStruct((N, 3), jnp.float32),
               jax.ShapeDtypeStruct((N, 7), jnp.float32)),
)


def kernel(x, edge_index, W_gcn, b_gcn, W_out, b_out):
    w8 = jnp.zeros((F, 8), jnp.float32).at[:, :3].set(W_gcn)

    hp = _hist_sc_kernel()(edge_index)
    xwT = _xw_tc(x, w8)
    yT3, y0, y1, y2, drow = _mid_tc(hp, xwT)
    ap = _msg_sc_kernel()(edge_index, y0, y1, y2)
    sel = jnp.tile(jnp.eye(3, dtype=jnp.float32), (1, NW))
    h, z = _fin_tc(ap, sel, yT3, drow,
                   b_gcn.reshape(1, 3), W_out, b_out.reshape(1, 7))
    return h, z


# speedup vs baseline: 155.5269x; 1.2164x over previous
import functools

import jax
import jax.numpy as jnp
from jax import lax
from jax.experimental import pallas as pl
from jax.experimental.pallas import tpu as pltpu
from jax.experimental.pallas import tpu_sc as plsc

N = 10000
E = 320000
F = 128
NC = 2
NS = 16
NW = NC * NS
EPW = E // NW
L = 16
UNROLL = 5
UNROLL_E = 25


@functools.cache
def _sc_mesh():
    return plsc.VectorSubcoreMesh(core_axis_name="c", subcore_axis_name="s",
                                  num_cores=NC, num_subcores=NS)


EW = EPW + 112


@functools.cache
def _hist_sc_kernel():
    return pl.kernel(
        _hist_sc_body,
        out_type=jax.ShapeDtypeStruct((NW, N), jnp.float32),
        mesh=_sc_mesh(),
        scratch_types=[
            pltpu.VMEM((2, EW), jnp.int32),
            pltpu.VMEM((N,), jnp.float32),
        ],
        compiler_params=pltpu.CompilerParams(needs_layout_passes=False),
    )


def _hist_sc_body(ei_hbm, out_hbm, ei_v, hist_v):
    wid = lax.axis_index("s") * NC + lax.axis_index("c")
    start = wid * EPW
    base = start // 128 * 128
    off = start - base
    pltpu.sync_copy(ei_hbm.at[:, pl.ds(base, EW)], ei_v)

    z16 = jnp.zeros((L,), jnp.float32)

    @plsc.parallel_loop(0, N // L, unroll=UNROLL)
    def _(i):
        hist_v[pl.ds(i * L, L)] = z16

    ones = jnp.ones((L,), jnp.float32)

    @plsc.parallel_loop(0, EPW // L, unroll=UNROLL_E)
    def _(i):
        t = ei_v[1, pl.ds(off + i * L, L)]
        plsc.addupdate_scatter(hist_v, [t], ones)

    pltpu.sync_copy(hist_v, out_hbm.at[wid])


@functools.cache
def _msg_sc_kernel():
    return pl.kernel(
        _msg_sc_body,
        out_type=jax.ShapeDtypeStruct((NW * 3, N), jnp.float32),
        mesh=_sc_mesh(),
        scratch_types=[
            pltpu.VMEM((2, EW), jnp.int32),
            pltpu.VMEM((N,), jnp.float32),
            pltpu.VMEM((N,), jnp.float32),
            pltpu.VMEM((N,), jnp.float32),
            pltpu.VMEM((N,), jnp.float32),
            pltpu.VMEM((N,), jnp.float32),
            pltpu.VMEM((N,), jnp.float32),
        ],
        compiler_params=pltpu.CompilerParams(needs_layout_passes=False),
    )


def _msg_sc_body(ei_hbm, y0_hbm, y1_hbm, y2_hbm, out_hbm,
                 ei_v, y0, y1, y2, a0, a1, a2):
    wid = lax.axis_index("s") * NC + lax.axis_index("c")
    start = wid * EPW
    base = start // 128 * 128
    off = start - base
    pltpu.sync_copy(ei_hbm.at[:, pl.ds(base, EW)], ei_v)
    pltpu.sync_copy(y0_hbm, y0)
    pltpu.sync_copy(y1_hbm, y1)
    pltpu.sync_copy(y2_hbm, y2)

    z16 = jnp.zeros((L,), jnp.float32)

    @plsc.parallel_loop(0, N // L, unroll=UNROLL)
    def _(i):
        a0[pl.ds(i * L, L)] = z16
        a1[pl.ds(i * L, L)] = z16
        a2[pl.ds(i * L, L)] = z16

    @plsc.parallel_loop(0, EPW // L, unroll=UNROLL_E)
    def _(i):
        s = ei_v[0, pl.ds(off + i * L, L)]
        t = ei_v[1, pl.ds(off + i * L, L)]
        plsc.addupdate_scatter(a0, [t], plsc.load_gather(y0, [s]))
        plsc.addupdate_scatter(a1, [t], plsc.load_gather(y1, [s]))
        plsc.addupdate_scatter(a2, [t], plsc.load_gather(y2, [s]))

    pltpu.sync_copy(a0, out_hbm.at[wid * 3 + 0])
    pltpu.sync_copy(a1, out_hbm.at[wid * 3 + 1])
    pltpu.sync_copy(a2, out_hbm.at[wid * 3 + 2])


def _xw_tc_body(x_ref, w8_ref, xwT_ref):
    xw = jnp.dot(x_ref[...], w8_ref[...],
                 preferred_element_type=jnp.float32)
    xwT_ref[...] = xw.T[0:3, :]


_xw_tc = pl.pallas_call(
    _xw_tc_body,
    out_shape=jax.ShapeDtypeStruct((3, N), jnp.float32),
)


def _mid_tc_body(hp_ref, xwT_ref, y_ref, y0_ref, y1_ref, y2_ref, d_ref):
    deg = jnp.sum(hp_ref[...], axis=0) + 1.0
    dv = lax.rsqrt(deg)
    yT = xwT_ref[...] * dv[None, :]
    y_ref[...] = yT
    y0_ref[...] = yT[0]
    y1_ref[...] = yT[1]
    y2_ref[...] = yT[2]
    d_ref[...] = dv[None, :]


_mid_tc = pl.pallas_call(
    _mid_tc_body,
    out_shape=(jax.ShapeDtypeStruct((3, N), jnp.float32),
               jax.ShapeDtypeStruct((N,), jnp.float32),
               jax.ShapeDtypeStruct((N,), jnp.float32),
               jax.ShapeDtypeStruct((N,), jnp.float32),
               jax.ShapeDtypeStruct((1, N), jnp.float32)),
)


def _fin_tc_body(ap_ref, sel_ref, y_ref, d_ref, b3_ref, wo_ref, bo_ref,
                 h_ref, z_ref):
    acc = jnp.dot(sel_ref[...], ap_ref[...],
                  preferred_element_type=jnp.float32)
    tot = (acc + y_ref[...]) * d_ref[...]
    hT = jnp.maximum(tot + b3_ref[...], 0.0)
    h_ref[...] = hT
    z_ref[...] = jnp.dot(wo_ref[...], hT,
                         preferred_element_type=jnp.float32) + bo_ref[...]


_fin_tc = pl.pallas_call(
    _fin_tc_body,
    out_shape=(jax.ShapeDtypeStruct((3, N), jnp.float32),
               jax.ShapeDtypeStruct((7, N), jnp.float32)),
)


def kernel(x, edge_index, W_gcn, b_gcn, W_out, b_out):
    w8 = jnp.zeros((F, 8), jnp.float32).at[:, :3].set(W_gcn)

    hp = _hist_sc_kernel()(edge_index)
    xwT = _xw_tc(x, w8)
    yT3, y0, y1, y2, drow = _mid_tc(hp, xwT)
    ap = _msg_sc_kernel()(edge_index, y0, y1, y2)
    sel = jnp.tile(jnp.eye(3, dtype=jnp.float32), (1, NW))
    hT, zT = _fin_tc(ap, sel, yT3, drow,
                     b_gcn.reshape(3, 1), W_out.T, b_out.reshape(7, 1))
    return hT.T, zT.T


# speedup vs baseline: 161.9013x; 1.0410x over previous
import functools

import jax
import jax.numpy as jnp
from jax import lax
from jax.experimental import pallas as pl
from jax.experimental.pallas import tpu as pltpu
from jax.experimental.pallas import tpu_sc as plsc

N = 10000
E = 320000
F = 128
NC = 2
NS = 16
NW = NC * NS
EPW = E // NW
L = 16
UNROLL = 5
UNROLL_E = 25
EW = EPW + 112


@functools.cache
def _sc_mesh():
    return plsc.VectorSubcoreMesh(core_axis_name="c", subcore_axis_name="s",
                                  num_cores=NC, num_subcores=NS)


@functools.cache
def _hist_sc_kernel():
    return pl.kernel(
        _hist_sc_body,
        out_type=jax.ShapeDtypeStruct((NW, N), jnp.float32),
        mesh=_sc_mesh(),
        scratch_types=[
            pltpu.VMEM((2, EW), jnp.int32),
            pltpu.VMEM((N,), jnp.float32),
            pltpu.SemaphoreType.DMA,
        ],
        compiler_params=pltpu.CompilerParams(needs_layout_passes=False),
    )


def _hist_sc_body(ei_hbm, out_hbm, ei_v, hist_v, sem):
    wid = lax.axis_index("s") * NC + lax.axis_index("c")
    start = wid * EPW
    base = start // 128 * 128
    off = start - base
    cp = pltpu.async_copy(ei_hbm.at[:, pl.ds(base, EW)], ei_v, sem)

    z16 = jnp.zeros((L,), jnp.float32)

    @plsc.parallel_loop(0, N // L, unroll=UNROLL)
    def _(i):
        hist_v[pl.ds(i * L, L)] = z16

    cp.wait()
    ones = jnp.ones((L,), jnp.float32)

    @plsc.parallel_loop(0, EPW // L, unroll=UNROLL_E)
    def _(i):
        t = ei_v[1, pl.ds(off + i * L, L)]
        plsc.addupdate_scatter(hist_v, [t], ones)

    pltpu.sync_copy(hist_v, out_hbm.at[wid])


@functools.cache
def _msg_sc_kernel():
    return pl.kernel(
        _msg_sc_body,
        out_type=jax.ShapeDtypeStruct((NW * 3, N), jnp.float32),
        mesh=_sc_mesh(),
        scratch_types=[
            pltpu.VMEM((2, EW), jnp.int32),
            pltpu.VMEM((3 * N,), jnp.float32),
            pltpu.VMEM((N,), jnp.float32),
            pltpu.VMEM((N,), jnp.float32),
            pltpu.VMEM((N,), jnp.float32),
            pltpu.SemaphoreType.DMA,
            pltpu.SemaphoreType.DMA,
        ],
        compiler_params=pltpu.CompilerParams(needs_layout_passes=False),
    )


def _msg_sc_body(ei_hbm, y_hbm, out_hbm,
                 ei_v, y_v, a0, a1, a2, sem0, sem1):
    wid = lax.axis_index("s") * NC + lax.axis_index("c")
    start = wid * EPW
    base = start // 128 * 128
    off = start - base
    cp0 = pltpu.async_copy(ei_hbm.at[:, pl.ds(base, EW)], ei_v, sem0)
    cp1 = pltpu.async_copy(y_hbm, y_v, sem1)

    z16 = jnp.zeros((L,), jnp.float32)

    @plsc.parallel_loop(0, N // L, unroll=UNROLL)
    def _(i):
        a0[pl.ds(i * L, L)] = z16
        a1[pl.ds(i * L, L)] = z16
        a2[pl.ds(i * L, L)] = z16

    cp0.wait()
    cp1.wait()

    n1 = jnp.full((L,), N, jnp.int32)
    n2 = jnp.full((L,), 2 * N, jnp.int32)

    @plsc.parallel_loop(0, EPW // L, unroll=UNROLL_E)
    def _(i):
        s = ei_v[0, pl.ds(off + i * L, L)]
        t = ei_v[1, pl.ds(off + i * L, L)]
        plsc.addupdate_scatter(a0, [t], plsc.load_gather(y_v, [s]))
        plsc.addupdate_scatter(a1, [t], plsc.load_gather(y_v, [s + n1]))
        plsc.addupdate_scatter(a2, [t], plsc.load_gather(y_v, [s + n2]))

    pltpu.sync_copy(a0, out_hbm.at[wid * 3 + 0])
    pltpu.sync_copy(a1, out_hbm.at[wid * 3 + 1])
    pltpu.sync_copy(a2, out_hbm.at[wid * 3 + 2])


def _xw_tc_body(x_ref, w8_ref, xwT_ref):
    xw = jnp.dot(x_ref[...], w8_ref[...],
                 preferred_element_type=jnp.float32)
    xwT_ref[...] = xw.T[0:3, :]


_xw_tc = pl.pallas_call(
    _xw_tc_body,
    out_shape=jax.ShapeDtypeStruct((3, N), jnp.float32),
)


def _mid_tc_body(hp_ref, xwT_ref, y_ref, yf_ref, d_ref):
    deg = jnp.sum(hp_ref[...], axis=0) + 1.0
    dv = lax.rsqrt(deg)
    yT = xwT_ref[...] * dv[None, :]
    y_ref[...] = yT
    yf_ref[pl.ds(0, N)] = yT[0]
    yf_ref[pl.ds(N, N)] = yT[1]
    yf_ref[pl.ds(2 * N, N)] = yT[2]
    d_ref[...] = dv[None, :]


_mid_tc = pl.pallas_call(
    _mid_tc_body,
    out_shape=(jax.ShapeDtypeStruct((3, N), jnp.float32),
               jax.ShapeDtypeStruct((3 * N,), jnp.float32),
               jax.ShapeDtypeStruct((1, N), jnp.float32)),
)


def _fin_tc_body(ap_ref, sel_ref, y_ref, d_ref, b3_ref, wo_ref, bo_ref,
                 h_ref, z_ref):
    acc = jnp.dot(sel_ref[...], ap_ref[...],
                  preferred_element_type=jnp.float32)
    tot = (acc + y_ref[...]) * d_ref[...]
    hT = jnp.maximum(tot + b3_ref[...], 0.0)
    h_ref[...] = hT
    z_ref[...] = jnp.dot(wo_ref[...], hT,
                         preferred_element_type=jnp.float32) + bo_ref[...]


_fin_tc = pl.pallas_call(
    _fin_tc_body,
    out_shape=(jax.ShapeDtypeStruct((3, N), jnp.float32),
               jax.ShapeDtypeStruct((7, N), jnp.float32)),
)


def kernel(x, edge_index, W_gcn, b_gcn, W_out, b_out):
    w8 = jnp.zeros((F, 8), jnp.float32).at[:, :3].set(W_gcn)

    hp = _hist_sc_kernel()(edge_index)
    xwT = _xw_tc(x, w8)
    yT3, yf, drow = _mid_tc(hp, xwT)
    ap = _msg_sc_kernel()(edge_index, yf)
    sel = jnp.tile(jnp.eye(3, dtype=jnp.float32), (1, NW))
    hT, zT = _fin_tc(ap, sel, yT3, drow,
                     b_gcn.reshape(3, 1), W_out.T, b_out.reshape(7, 1))
    return hT.T, zT.T


# speedup vs baseline: 165.4045x; 1.0216x over previous
import functools

import jax
import jax.numpy as jnp
from jax import lax
from jax.experimental import pallas as pl
from jax.experimental.pallas import tpu as pltpu
from jax.experimental.pallas import tpu_sc as plsc

N = 10000
E = 320000
F = 128
NC = 2
NS = 16
NW = NC * NS
EPW = E // NW
L = 16
UNROLL = 5
UNROLL_E = 125
EW = EPW + 112


@functools.cache
def _sc_mesh():
    return plsc.VectorSubcoreMesh(core_axis_name="c", subcore_axis_name="s",
                                  num_cores=NC, num_subcores=NS)


@functools.cache
def _hist_sc_kernel():
    return pl.kernel(
        _hist_sc_body,
        out_type=jax.ShapeDtypeStruct((NW, N), jnp.float32),
        mesh=_sc_mesh(),
        scratch_types=[
            pltpu.VMEM((2, EW), jnp.int32),
            pltpu.VMEM((N,), jnp.float32),
            pltpu.SemaphoreType.DMA,
        ],
        compiler_params=pltpu.CompilerParams(needs_layout_passes=False),
    )


def _hist_sc_body(ei_hbm, out_hbm, ei_v, hist_v, sem):
    wid = lax.axis_index("s") * NC + lax.axis_index("c")
    start = wid * EPW
    base = start // 128 * 128
    off = start - base
    cp = pltpu.async_copy(ei_hbm.at[:, pl.ds(base, EW)], ei_v, sem)

    z16 = jnp.zeros((L,), jnp.float32)

    @plsc.parallel_loop(0, N // L, unroll=UNROLL)
    def _(i):
        hist_v[pl.ds(i * L, L)] = z16

    cp.wait()
    ones = jnp.ones((L,), jnp.float32)

    @plsc.parallel_loop(0, EPW // L, unroll=UNROLL_E)
    def _(i):
        t = ei_v[1, pl.ds(off + i * L, L)]
        plsc.addupdate_scatter(hist_v, [t], ones)

    pltpu.sync_copy(hist_v, out_hbm.at[wid])


@functools.cache
def _msg_sc_kernel():
    return pl.kernel(
        _msg_sc_body,
        out_type=jax.ShapeDtypeStruct((NW * 3, N), jnp.float32),
        mesh=_sc_mesh(),
        scratch_types=[
            pltpu.VMEM((2, EW), jnp.int32),
            pltpu.VMEM((3 * N,), jnp.float32),
            pltpu.VMEM((N,), jnp.float32),
            pltpu.VMEM((N,), jnp.float32),
            pltpu.VMEM((N,), jnp.float32),
            pltpu.SemaphoreType.DMA,
            pltpu.SemaphoreType.DMA,
        ],
        compiler_params=pltpu.CompilerParams(needs_layout_passes=False),
    )


def _msg_sc_body(ei_hbm, y_hbm, out_hbm,
                 ei_v, y_v, a0, a1, a2, sem0, sem1):
    wid = lax.axis_index("s") * NC + lax.axis_index("c")
    start = wid * EPW
    base = start // 128 * 128
    off = start - base
    cp0 = pltpu.async_copy(ei_hbm.at[:, pl.ds(base, EW)], ei_v, sem0)
    cp1 = pltpu.async_copy(y_hbm, y_v, sem1)

    z16 = jnp.zeros((L,), jnp.float32)

    @plsc.parallel_loop(0, N // L, unroll=UNROLL)
    def _(i):
        a0[pl.ds(i * L, L)] = z16
        a1[pl.ds(i * L, L)] = z16
        a2[pl.ds(i * L, L)] = z16

    cp0.wait()
    cp1.wait()

    n1 = jnp.full((L,), N, jnp.int32)
    n2 = jnp.full((L,), 2 * N, jnp.int32)

    @plsc.parallel_loop(0, EPW // L, unroll=UNROLL_E)
    def _(i):
        s = ei_v[0, pl.ds(off + i * L, L)]
        t = ei_v[1, pl.ds(off + i * L, L)]
        plsc.addupdate_scatter(a0, [t], plsc.load_gather(y_v, [s]))
        plsc.addupdate_scatter(a1, [t], plsc.load_gather(y_v, [s + n1]))
        plsc.addupdate_scatter(a2, [t], plsc.load_gather(y_v, [s + n2]))

    pltpu.sync_copy(a0, out_hbm.at[wid * 3 + 0])
    pltpu.sync_copy(a1, out_hbm.at[wid * 3 + 1])
    pltpu.sync_copy(a2, out_hbm.at[wid * 3 + 2])


def _xw_tc_body(x_ref, w8_ref, xwT_ref):
    xw = jnp.dot(x_ref[...], w8_ref[...],
                 preferred_element_type=jnp.float32)
    xwT_ref[...] = xw.T[0:3, :]


_xw_tc = pl.pallas_call(
    _xw_tc_body,
    out_shape=jax.ShapeDtypeStruct((3, N), jnp.float32),
)


def _mid_tc_body(hp_ref, xwT_ref, y_ref, yf_ref, d_ref):
    deg = jnp.sum(hp_ref[...], axis=0) + 1.0
    dv = lax.rsqrt(deg)
    yT = xwT_ref[...] * dv[None, :]
    y_ref[...] = yT
    yf_ref[pl.ds(0, N)] = yT[0]
    yf_ref[pl.ds(N, N)] = yT[1]
    yf_ref[pl.ds(2 * N, N)] = yT[2]
    d_ref[...] = dv[None, :]


_mid_tc = pl.pallas_call(
    _mid_tc_body,
    out_shape=(jax.ShapeDtypeStruct((3, N), jnp.float32),
               jax.ShapeDtypeStruct((3 * N,), jnp.float32),
               jax.ShapeDtypeStruct((1, N), jnp.float32)),
)


def _fin_tc_body(ap_ref, sel_ref, y_ref, d_ref, b3_ref, wo_ref, bo_ref,
                 h_ref, z_ref):
    acc = jnp.dot(sel_ref[...], ap_ref[...],
                  preferred_element_type=jnp.float32)
    tot = (acc + y_ref[...]) * d_ref[...]
    hT = jnp.maximum(tot + b3_ref[...], 0.0)
    h_ref[...] = hT
    z_ref[...] = jnp.dot(wo_ref[...], hT,
                         preferred_element_type=jnp.float32) + bo_ref[...]


_fin_tc = pl.pallas_call(
    _fin_tc_body,
    out_shape=(jax.ShapeDtypeStruct((3, N), jnp.float32),
               jax.ShapeDtypeStruct((7, N), jnp.float32)),
)


def kernel(x, edge_index, W_gcn, b_gcn, W_out, b_out):
    w8 = jnp.zeros((F, 8), jnp.float32).at[:, :3].set(W_gcn)

    hp = _hist_sc_kernel()(edge_index)
    xwT = _xw_tc(x, w8)
    yT3, yf, drow = _mid_tc(hp, xwT)
    ap = _msg_sc_kernel()(edge_index, yf)
    sel = jnp.tile(jnp.eye(3, dtype=jnp.float32), (1, NW))
    hT, zT = _fin_tc(ap, sel, yT3, drow,
                     b_gcn.reshape(3, 1), W_out.T, b_out.reshape(7, 1))
    return hT.T, zT.T
